# baseline (device time: 203855 ns/iter reference)
import jax
import jax.numpy as jnp
from jax import lax
from jax.experimental import pallas as pl
from jax.experimental.pallas import tpu as pltpu

N_DEV = 16
NSEG = 8


def _silu(y):
    return y * (1.0 / (1.0 + jnp.exp(-y)))


def kernel(x, w_mat):
    m_per, k = x.shape
    n_per = w_mat.shape[1]
    half = m_per // 2
    seg = half // NSEG
    x = x.astype(jnp.bfloat16)
    w_mat = w_mat.astype(jnp.bfloat16)

    def body(x_ref, w_ref, out_ref, cw_buf, ccw_buf,
             cw_send, cw_recv, ccw_send, ccw_recv):
        my = lax.axis_index("i")
        left = lax.rem(my + N_DEV - 1, N_DEV)
        right = lax.rem(my + 1, N_DEV)

        barrier_sem = pltpu.get_barrier_semaphore()
        for nbr in [left, right]:
            pl.semaphore_signal(
                barrier_sem, inc=1,
                device_id=(nbr,), device_id_type=pl.DeviceIdType.MESH,
            )
        pl.semaphore_wait(barrier_sem, 2)

        cw_buf[0, :, :] = x_ref[:half, :]
        ccw_buf[0, :, :] = x_ref[half:, :]

        def make(buf, send_sems, recv_sems, h, s, dst):
            return pltpu.make_async_remote_copy(
                src_ref=buf.at[h, pl.ds(s * seg, seg)],
                dst_ref=buf.at[h + 1, pl.ds(s * seg, seg)],
                send_sem=send_sems.at[h, s],
                recv_sem=recv_sems.at[h, s],
                device_id=(dst,), device_id_type=pl.DeviceIdType.MESH,
            )

        cw_rdma = [[make(cw_buf, cw_send, cw_recv, h, s, right)
                    for s in range(NSEG)] for h in range(N_DEV - 1)]
        ccw_rdma = [[make(ccw_buf, ccw_send, ccw_recv, h, s, left)
                     for s in range(NSEG)] for h in range(N_DEV - 1)]

        for s in range(NSEG):
            cw_rdma[0][s].start()
            ccw_rdma[0][s].start()

        y = jnp.dot(x_ref[:, :], w_ref[:, :], preferred_element_type=jnp.float32)
        out_ref[pl.ds(my * m_per, m_per), :] = _silu(y)

        for h in range(N_DEV - 1):
            for s in range(NSEG):
                cw_rdma[h][s].wait_recv()
                if h + 1 < N_DEV - 1:
                    cw_rdma[h + 1][s].start()
                ccw_rdma[h][s].wait_recv()
                if h + 1 < N_DEV - 1:
                    ccw_rdma[h + 1][s].start()

            cw_origin = lax.rem(my + N_DEV - 1 - h, N_DEV)
            y = jnp.dot(cw_buf[h + 1, :, :], w_ref[:, :],
                        preferred_element_type=jnp.float32)
            out_ref[pl.ds(cw_origin * m_per, half), :] = _silu(y)

            ccw_origin = lax.rem(my + h + 1, N_DEV)
            y = jnp.dot(ccw_buf[h + 1, :, :], w_ref[:, :],
                        preferred_element_type=jnp.float32)
            out_ref[pl.ds(ccw_origin * m_per + half, half), :] = _silu(y)

        for h in range(N_DEV - 1):
            for s in range(NSEG):
                cw_rdma[h][s].wait_send()
                ccw_rdma[h][s].wait_send()

    return pl.pallas_call(
        body,
        out_shape=jax.ShapeDtypeStruct((N_DEV * m_per, n_per), jnp.float32),
        in_specs=[
            pl.BlockSpec(memory_space=pltpu.VMEM),
            pl.BlockSpec(memory_space=pltpu.VMEM),
        ],
        out_specs=pl.BlockSpec(memory_space=pltpu.VMEM),
        scratch_shapes=[
            pltpu.VMEM((N_DEV, half, k), jnp.bfloat16),
            pltpu.VMEM((N_DEV, half, k), jnp.bfloat16),
            pltpu.SemaphoreType.DMA((N_DEV - 1, NSEG)),
            pltpu.SemaphoreType.DMA((N_DEV - 1, NSEG)),
            pltpu.SemaphoreType.DMA((N_DEV - 1, NSEG)),
            pltpu.SemaphoreType.DMA((N_DEV - 1, NSEG)),
        ],
        compiler_params=pltpu.CompilerParams(
            collective_id=0,
            vmem_limit_bytes=63 * 1024 * 1024,
        ),
    )(x, w_mat)


# device time: 200382 ns/iter; 1.0173x vs baseline; 1.0173x over previous
import jax
import jax.numpy as jnp
from jax import lax
from jax.experimental import pallas as pl
from jax.experimental.pallas import tpu as pltpu

N_DEV = 16
FAR = N_DEV // 2
NEAR = FAR - 1
NSEG = 2


def _silu(y):
    return y * (1.0 / (1.0 + jnp.exp(-y)))


def kernel(x, w_mat):
    m_per, k = x.shape
    n_per = w_mat.shape[1]
    half = m_per // 2
    seg = half // NSEG
    x = x.astype(jnp.bfloat16)
    w_mat = w_mat.astype(jnp.bfloat16)

    def body(x_ref, w_ref, out_ref,
             cwA_buf, cwB_buf, ccwB_buf, ccwA_buf,
             cwA_send, cwA_recv, cwB_send, cwB_recv,
             ccwB_send, ccwB_recv, ccwA_send, ccwA_recv):
        my = lax.axis_index("i")
        left = lax.rem(my + N_DEV - 1, N_DEV)
        right = lax.rem(my + 1, N_DEV)

        barrier_sem = pltpu.get_barrier_semaphore()
        for nbr in [left, right]:
            pl.semaphore_signal(
                barrier_sem, inc=1,
                device_id=(nbr,), device_id_type=pl.DeviceIdType.MESH,
            )
        pl.semaphore_wait(barrier_sem, 2)

        def make(buf, send_sems, recv_sems, h, s, dst, row_off):
            src = (x_ref.at[pl.ds(row_off + s * seg, seg)] if h == 0
                   else buf.at[h, pl.ds(s * seg, seg)])
            return pltpu.make_async_remote_copy(
                src_ref=src,
                dst_ref=buf.at[h + 1, pl.ds(s * seg, seg)],
                send_sem=send_sems.at[h, s],
                recv_sem=recv_sems.at[h, s],
                device_id=(dst,), device_id_type=pl.DeviceIdType.MESH,
            )

        cwA = [[make(cwA_buf, cwA_send, cwA_recv, h, s, right, 0)
                for s in range(NSEG)] for h in range(FAR)]
        cwB = [[make(cwB_buf, cwB_send, cwB_recv, h, s, right, half)
                for s in range(NSEG)] for h in range(NEAR)]
        ccwB = [[make(ccwB_buf, ccwB_send, ccwB_recv, h, s, left, half)
                 for s in range(NSEG)] for h in range(FAR)]
        ccwA = [[make(ccwA_buf, ccwA_send, ccwA_recv, h, s, left, 0)
                 for s in range(NSEG)] for h in range(NEAR)]

        for s in range(NSEG):
            cwA[0][s].start()
            ccwB[0][s].start()
            cwB[0][s].start()
            ccwA[0][s].start()

        y = jnp.dot(x_ref[:, :], w_ref[:, :], preferred_element_type=jnp.float32)
        out_ref[pl.ds(my * m_per, m_per), :] = _silu(y)

        for h in range(FAR):
            for s in range(NSEG):
                cwA[h][s].wait_recv()
                if h + 1 < FAR:
                    cwA[h + 1][s].start()
                ccwB[h][s].wait_recv()
                if h + 1 < FAR:
                    ccwB[h + 1][s].start()
                if h < NEAR:
                    cwB[h][s].wait_recv()
                    if h + 1 < NEAR:
                        cwB[h + 1][s].start()
                    ccwA[h][s].wait_recv()
                    if h + 1 < NEAR:
                        ccwA[h + 1][s].start()

            d = h + 1
            cw_origin = lax.rem(my + N_DEV - d, N_DEV)
            y = jnp.dot(cwA_buf[d, :, :], w_ref[:, :],
                        preferred_element_type=jnp.float32)
            out_ref[pl.ds(cw_origin * m_per, half), :] = _silu(y)
            if d <= NEAR:
                y = jnp.dot(cwB_buf[d, :, :], w_ref[:, :],
                            preferred_element_type=jnp.float32)
                out_ref[pl.ds(cw_origin * m_per + half, half), :] = _silu(y)
            ccw_origin = lax.rem(my + d, N_DEV)
            y = jnp.dot(ccwB_buf[d, :, :], w_ref[:, :],
                        preferred_element_type=jnp.float32)
            out_ref[pl.ds(ccw_origin * m_per + half, half), :] = _silu(y)
            if d <= NEAR:
                y = jnp.dot(ccwA_buf[d, :, :], w_ref[:, :],
                            preferred_element_type=jnp.float32)
                out_ref[pl.ds(ccw_origin * m_per, half), :] = _silu(y)

        for h in range(FAR):
            for s in range(NSEG):
                cwA[h][s].wait_send()
                ccwB[h][s].wait_send()
                if h < NEAR:
                    cwB[h][s].wait_send()
                    ccwA[h][s].wait_send()

    return pl.pallas_call(
        body,
        out_shape=jax.ShapeDtypeStruct((N_DEV * m_per, n_per), jnp.float32),
        in_specs=[
            pl.BlockSpec(memory_space=pltpu.VMEM),
            pl.BlockSpec(memory_space=pltpu.VMEM),
        ],
        out_specs=pl.BlockSpec(memory_space=pltpu.VMEM),
        scratch_shapes=[
            pltpu.VMEM((FAR + 1, half, k), jnp.bfloat16),
            pltpu.VMEM((NEAR + 1, half, k), jnp.bfloat16),
            pltpu.VMEM((FAR + 1, half, k), jnp.bfloat16),
            pltpu.VMEM((NEAR + 1, half, k), jnp.bfloat16),
            pltpu.SemaphoreType.DMA((FAR, NSEG)),
            pltpu.SemaphoreType.DMA((FAR, NSEG)),
            pltpu.SemaphoreType.DMA((NEAR, NSEG)),
            pltpu.SemaphoreType.DMA((NEAR, NSEG)),
            pltpu.SemaphoreType.DMA((FAR, NSEG)),
            pltpu.SemaphoreType.DMA((FAR, NSEG)),
            pltpu.SemaphoreType.DMA((NEAR, NSEG)),
            pltpu.SemaphoreType.DMA((NEAR, NSEG)),
        ],
        compiler_params=pltpu.CompilerParams(
            collective_id=0,
            vmem_limit_bytes=63 * 1024 * 1024,
        ),
    )(x, w_mat)


# device time: 193458 ns/iter; 1.0537x vs baseline; 1.0358x over previous
import jax
import jax.numpy as jnp
from jax import lax
from jax.experimental import pallas as pl
from jax.experimental.pallas import tpu as pltpu

N_DEV = 16
FAR = N_DEV // 2
NEAR = FAR - 1
NSEG = 2


def _silu(y):
    return y * (1.0 / (1.0 + jnp.exp(-y)))


def kernel(x, w_mat):
    m_per, k = x.shape
    n_per = w_mat.shape[1]
    half = m_per // 2
    seg = half // NSEG

    def body(x_f32_ref, w_f32_ref, out_ref,
             x_ref, w_ref,
             cwA_buf, cwB_buf, ccwB_buf, ccwA_buf,
             cwA_send, cwA_recv, cwB_send, cwB_recv,
             ccwB_send, ccwB_recv, ccwA_send, ccwA_recv):
        my = lax.axis_index("i")
        left = lax.rem(my + N_DEV - 1, N_DEV)
        right = lax.rem(my + 1, N_DEV)

        barrier_sem = pltpu.get_barrier_semaphore()
        for nbr in [left, right]:
            pl.semaphore_signal(
                barrier_sem, inc=1,
                device_id=(nbr,), device_id_type=pl.DeviceIdType.MESH,
            )
        pl.semaphore_wait(barrier_sem, 2)

        x_ref[:, :] = x_f32_ref[:, :].astype(jnp.bfloat16)

        def make(buf, send_sems, recv_sems, h, s, dst, row_off):
            src = (x_ref.at[pl.ds(row_off + s * seg, seg)] if h == 0
                   else buf.at[h, pl.ds(s * seg, seg)])
            return pltpu.make_async_remote_copy(
                src_ref=src,
                dst_ref=buf.at[h + 1, pl.ds(s * seg, seg)],
                send_sem=send_sems.at[h, s],
                recv_sem=recv_sems.at[h, s],
                device_id=(dst,), device_id_type=pl.DeviceIdType.MESH,
            )

        cwA = [[make(cwA_buf, cwA_send, cwA_recv, h, s, right, 0)
                for s in range(NSEG)] for h in range(FAR)]
        cwB = [[make(cwB_buf, cwB_send, cwB_recv, h, s, right, half)
                for s in range(NSEG)] for h in range(NEAR)]
        ccwB = [[make(ccwB_buf, ccwB_send, ccwB_recv, h, s, left, half)
                 for s in range(NSEG)] for h in range(FAR)]
        ccwA = [[make(ccwA_buf, ccwA_send, ccwA_recv, h, s, left, 0)
                 for s in range(NSEG)] for h in range(NEAR)]

        for s in range(NSEG):
            cwA[0][s].start()
            ccwB[0][s].start()
            cwB[0][s].start()
            ccwA[0][s].start()

        w_ref[:, :] = w_f32_ref[:, :].astype(jnp.bfloat16)

        y = jnp.dot(x_ref[:, :], w_ref[:, :], preferred_element_type=jnp.float32)
        out_ref[pl.ds(my * m_per, m_per), :] = _silu(y)

        for h in range(FAR):
            for s in range(NSEG):
                cwA[h][s].wait_recv()
                if h + 1 < FAR:
                    cwA[h + 1][s].start()
                ccwB[h][s].wait_recv()
                if h + 1 < FAR:
                    ccwB[h + 1][s].start()
                if h < NEAR:
                    cwB[h][s].wait_recv()
                    if h + 1 < NEAR:
                        cwB[h + 1][s].start()
                    ccwA[h][s].wait_recv()
                    if h + 1 < NEAR:
                        ccwA[h + 1][s].start()

            d = h + 1
            cw_origin = lax.rem(my + N_DEV - d, N_DEV)
            y = jnp.dot(cwA_buf[d, :, :], w_ref[:, :],
                        preferred_element_type=jnp.float32)
            out_ref[pl.ds(cw_origin * m_per, half), :] = _silu(y)
            if d <= NEAR:
                y = jnp.dot(cwB_buf[d, :, :], w_ref[:, :],
                            preferred_element_type=jnp.float32)
                out_ref[pl.ds(cw_origin * m_per + half, half), :] = _silu(y)
            ccw_origin = lax.rem(my + d, N_DEV)
            y = jnp.dot(ccwB_buf[d, :, :], w_ref[:, :],
                        preferred_element_type=jnp.float32)
            out_ref[pl.ds(ccw_origin * m_per + half, half), :] = _silu(y)
            if d <= NEAR:
                y = jnp.dot(ccwA_buf[d, :, :], w_ref[:, :],
                            preferred_element_type=jnp.float32)
                out_ref[pl.ds(ccw_origin * m_per, half), :] = _silu(y)

        for h in range(FAR):
            for s in range(NSEG):
                cwA[h][s].wait_send()
                ccwB[h][s].wait_send()
                if h < NEAR:
                    cwB[h][s].wait_send()
                    ccwA[h][s].wait_send()

    return pl.pallas_call(
        body,
        out_shape=jax.ShapeDtypeStruct((N_DEV * m_per, n_per), jnp.float32),
        in_specs=[
            pl.BlockSpec(memory_space=pltpu.VMEM),
            pl.BlockSpec(memory_space=pltpu.VMEM),
        ],
        out_specs=pl.BlockSpec(memory_space=pltpu.VMEM),
        scratch_shapes=[
            pltpu.VMEM((m_per, k), jnp.bfloat16),
            pltpu.VMEM((k, n_per), jnp.bfloat16),
            pltpu.VMEM((FAR + 1, half, k), jnp.bfloat16),
            pltpu.VMEM((NEAR + 1, half, k), jnp.bfloat16),
            pltpu.VMEM((FAR + 1, half, k), jnp.bfloat16),
            pltpu.VMEM((NEAR + 1, half, k), jnp.bfloat16),
            pltpu.SemaphoreType.DMA((FAR, NSEG)),
            pltpu.SemaphoreType.DMA((FAR, NSEG)),
            pltpu.SemaphoreType.DMA((NEAR, NSEG)),
            pltpu.SemaphoreType.DMA((NEAR, NSEG)),
            pltpu.SemaphoreType.DMA((FAR, NSEG)),
            pltpu.SemaphoreType.DMA((FAR, NSEG)),
            pltpu.SemaphoreType.DMA((NEAR, NSEG)),
            pltpu.SemaphoreType.DMA((NEAR, NSEG)),
        ],
        compiler_params=pltpu.CompilerParams(
            collective_id=0,
            vmem_limit_bytes=63 * 1024 * 1024,
        ),
    )(x, w_mat)


# device time: 191210 ns/iter; 1.0661x vs baseline; 1.0118x over previous
import jax
import jax.numpy as jnp
from jax import lax
from jax.experimental import pallas as pl
from jax.experimental.pallas import tpu as pltpu

N_DEV = 16
FAR = N_DEV // 2
NEAR = FAR - 1
NSEG = 2


def _silu(y):
    return y * (1.0 / (1.0 + jnp.exp(-y)))


def kernel(x, w_mat):
    m_per, k = x.shape
    n_per = w_mat.shape[1]
    half = m_per // 2
    seg = half // NSEG

    def body(x_f32_ref, w_f32_ref, out_ref,
             x_ref, w_ref, stage, stage_sems,
             cwA_buf, cwB_buf, ccwB_buf, ccwA_buf,
             cwA_send, cwA_recv, cwB_send, cwB_recv,
             ccwB_send, ccwB_recv, ccwA_send, ccwA_recv):
        my = lax.axis_index("i")
        left = lax.rem(my + N_DEV - 1, N_DEV)
        right = lax.rem(my + 1, N_DEV)

        barrier_sem = pltpu.get_barrier_semaphore()
        for nbr in [left, right]:
            pl.semaphore_signal(
                barrier_sem, inc=1,
                device_id=(nbr,), device_id_type=pl.DeviceIdType.MESH,
            )
        pl.semaphore_wait(barrier_sem, 2)

        x_ref[:, :] = x_f32_ref[:, :].astype(jnp.bfloat16)

        def make(buf, send_sems, recv_sems, h, s, dst, row_off):
            src = (x_ref.at[pl.ds(row_off + s * seg, seg)] if h == 0
                   else buf.at[h, pl.ds(s * seg, seg)])
            return pltpu.make_async_remote_copy(
                src_ref=src,
                dst_ref=buf.at[h + 1, pl.ds(s * seg, seg)],
                send_sem=send_sems.at[h, s],
                recv_sem=recv_sems.at[h, s],
                device_id=(dst,), device_id_type=pl.DeviceIdType.MESH,
            )

        cwA = [[make(cwA_buf, cwA_send, cwA_recv, h, s, right, 0)
                for s in range(NSEG)] for h in range(FAR)]
        cwB = [[make(cwB_buf, cwB_send, cwB_recv, h, s, right, half)
                for s in range(NSEG)] for h in range(NEAR)]
        ccwB = [[make(ccwB_buf, ccwB_send, ccwB_recv, h, s, left, half)
                 for s in range(NSEG)] for h in range(FAR)]
        ccwA = [[make(ccwA_buf, ccwA_send, ccwA_recv, h, s, left, 0)
                 for s in range(NSEG)] for h in range(NEAR)]

        for s in range(NSEG):
            cwA[0][s].start()
            ccwB[0][s].start()
            cwB[0][s].start()
            ccwA[0][s].start()

        w_ref[:, :] = w_f32_ref[:, :].astype(jnp.bfloat16)

        n_stores = [0]
        pending = [None, None]

        def store_half(row_start, vals):
            slot = n_stores[0] % 2
            n_stores[0] += 1
            if pending[slot] is not None:
                pending[slot].wait()
            stage[slot, :, :] = vals
            cp = pltpu.make_async_copy(
                stage.at[slot],
                out_ref.at[pl.ds(row_start, half)],
                stage_sems.at[slot],
            )
            cp.start()
            pending[slot] = cp

        y = jnp.dot(x_ref[:, :], w_ref[:, :], preferred_element_type=jnp.float32)
        store_half(my * m_per, _silu(y[:half, :]))
        store_half(my * m_per + half, _silu(y[half:, :]))

        for h in range(FAR):
            for s in range(NSEG):
                cwA[h][s].wait_recv()
                if h + 1 < FAR:
                    cwA[h + 1][s].start()
                ccwB[h][s].wait_recv()
                if h + 1 < FAR:
                    ccwB[h + 1][s].start()
                if h < NEAR:
                    cwB[h][s].wait_recv()
                    if h + 1 < NEAR:
                        cwB[h + 1][s].start()
                    ccwA[h][s].wait_recv()
                    if h + 1 < NEAR:
                        ccwA[h + 1][s].start()

            d = h + 1
            cw_origin = lax.rem(my + N_DEV - d, N_DEV)
            y = jnp.dot(cwA_buf[d, :, :], w_ref[:, :],
                        preferred_element_type=jnp.float32)
            store_half(cw_origin * m_per, _silu(y))
            if d <= NEAR:
                y = jnp.dot(cwB_buf[d, :, :], w_ref[:, :],
                            preferred_element_type=jnp.float32)
                store_half(cw_origin * m_per + half, _silu(y))
            ccw_origin = lax.rem(my + d, N_DEV)
            y = jnp.dot(ccwB_buf[d, :, :], w_ref[:, :],
                        preferred_element_type=jnp.float32)
            store_half(ccw_origin * m_per + half, _silu(y))
            if d <= NEAR:
                y = jnp.dot(ccwA_buf[d, :, :], w_ref[:, :],
                            preferred_element_type=jnp.float32)
                store_half(ccw_origin * m_per, _silu(y))

        for h in range(FAR):
            for s in range(NSEG):
                cwA[h][s].wait_send()
                ccwB[h][s].wait_send()
                if h < NEAR:
                    cwB[h][s].wait_send()
                    ccwA[h][s].wait_send()
        for cp in pending:
            if cp is not None:
                cp.wait()

    return pl.pallas_call(
        body,
        out_shape=jax.ShapeDtypeStruct((N_DEV * m_per, n_per), jnp.float32),
        in_specs=[
            pl.BlockSpec(memory_space=pltpu.VMEM),
            pl.BlockSpec(memory_space=pltpu.VMEM),
        ],
        out_specs=pl.BlockSpec(memory_space=pl.ANY),
        scratch_shapes=[
            pltpu.VMEM((m_per, k), jnp.bfloat16),
            pltpu.VMEM((k, n_per), jnp.bfloat16),
            pltpu.VMEM((2, half, n_per), jnp.float32),
            pltpu.SemaphoreType.DMA((2,)),
            pltpu.VMEM((FAR + 1, half, k), jnp.bfloat16),
            pltpu.VMEM((NEAR + 1, half, k), jnp.bfloat16),
            pltpu.VMEM((FAR + 1, half, k), jnp.bfloat16),
            pltpu.VMEM((NEAR + 1, half, k), jnp.bfloat16),
            pltpu.SemaphoreType.DMA((FAR, NSEG)),
            pltpu.SemaphoreType.DMA((FAR, NSEG)),
            pltpu.SemaphoreType.DMA((NEAR, NSEG)),
            pltpu.SemaphoreType.DMA((NEAR, NSEG)),
            pltpu.SemaphoreType.DMA((FAR, NSEG)),
            pltpu.SemaphoreType.DMA((FAR, NSEG)),
            pltpu.SemaphoreType.DMA((NEAR, NSEG)),
            pltpu.SemaphoreType.DMA((NEAR, NSEG)),
        ],
        compiler_params=pltpu.CompilerParams(
            collective_id=0,
            vmem_limit_bytes=63 * 1024 * 1024,
        ),
    )(x, w_mat)


# device time: 190907 ns/iter; 1.0678x vs baseline; 1.0016x over previous
import jax
import jax.numpy as jnp
from jax import lax
from jax.experimental import pallas as pl
from jax.experimental.pallas import tpu as pltpu

N_DEV = 16
FAR = N_DEV // 2
NEAR = FAR - 1
NSEG = 4


def _silu(y):
    return y * (1.0 / (1.0 + jnp.exp(-y)))


def kernel(x, w_mat):
    m_per, k = x.shape
    n_per = w_mat.shape[1]
    half = m_per // 2
    seg = half // NSEG

    def body(x_f32_ref, w_f32_ref, out_ref,
             x_ref, w_ref, stage, stage_sems,
             cwA_buf, cwB_buf, ccwB_buf, ccwA_buf,
             cwA_send, cwA_recv, cwB_send, cwB_recv,
             ccwB_send, ccwB_recv, ccwA_send, ccwA_recv):
        my = lax.axis_index("i")
        left = lax.rem(my + N_DEV - 1, N_DEV)
        right = lax.rem(my + 1, N_DEV)

        barrier_sem = pltpu.get_barrier_semaphore()
        for nbr in [left, right]:
            pl.semaphore_signal(
                barrier_sem, inc=1,
                device_id=(nbr,), device_id_type=pl.DeviceIdType.MESH,
            )
        pl.semaphore_wait(barrier_sem, 2)

        def make(buf, send_sems, recv_sems, h, s, dst, row_off):
            src = (x_ref.at[pl.ds(row_off + s * seg, seg)] if h == 0
                   else buf.at[h, pl.ds(s * seg, seg)])
            return pltpu.make_async_remote_copy(
                src_ref=src,
                dst_ref=buf.at[h + 1, pl.ds(s * seg, seg)],
                send_sem=send_sems.at[h, s],
                recv_sem=recv_sems.at[h, s],
                device_id=(dst,), device_id_type=pl.DeviceIdType.MESH,
            )

        cwA = [[make(cwA_buf, cwA_send, cwA_recv, h, s, right, 0)
                for s in range(NSEG)] for h in range(FAR)]
        cwB = [[make(cwB_buf, cwB_send, cwB_recv, h, s, right, half)
                for s in range(NSEG)] for h in range(NEAR)]
        ccwB = [[make(ccwB_buf, ccwB_send, ccwB_recv, h, s, left, half)
                 for s in range(NSEG)] for h in range(FAR)]
        ccwA = [[make(ccwA_buf, ccwA_send, ccwA_recv, h, s, left, 0)
                 for s in range(NSEG)] for h in range(NEAR)]

        for s in range(NSEG):
            x_ref[s * seg:(s + 1) * seg, :] = (
                x_f32_ref[s * seg:(s + 1) * seg, :].astype(jnp.bfloat16))
            x_ref[half + s * seg:half + (s + 1) * seg, :] = (
                x_f32_ref[half + s * seg:half + (s + 1) * seg, :]
                .astype(jnp.bfloat16))
            cwA[0][s].start()
            ccwB[0][s].start()
            cwB[0][s].start()
            ccwA[0][s].start()

        w_ref[:, :] = w_f32_ref[:, :].astype(jnp.bfloat16)

        n_stores = [0]
        pending = [None, None]

        def store_half(row_start, vals):
            slot = n_stores[0] % 2
            n_stores[0] += 1
            if pending[slot] is not None:
                pending[slot].wait()
            stage[slot, :, :] = vals
            cp = pltpu.make_async_copy(
                stage.at[slot],
                out_ref.at[pl.ds(row_start, half)],
                stage_sems.at[slot],
            )
            cp.start()
            pending[slot] = cp

        y = jnp.dot(x_ref[:, :], w_ref[:, :], preferred_element_type=jnp.float32)
        store_half(my * m_per, _silu(y[:half, :]))
        store_half(my * m_per + half, _silu(y[half:, :]))

        for h in range(FAR):
            for s in range(NSEG):
                cwA[h][s].wait_recv()
                if h + 1 < FAR:
                    cwA[h + 1][s].start()
                ccwB[h][s].wait_recv()
                if h + 1 < FAR:
                    ccwB[h + 1][s].start()
                if h < NEAR:
                    cwB[h][s].wait_recv()
                    if h + 1 < NEAR:
                        cwB[h + 1][s].start()
                    ccwA[h][s].wait_recv()
                    if h + 1 < NEAR:
                        ccwA[h + 1][s].start()

            d = h + 1
            cw_origin = lax.rem(my + N_DEV - d, N_DEV)
            y = jnp.dot(cwA_buf[d, :, :], w_ref[:, :],
                        preferred_element_type=jnp.float32)
            store_half(cw_origin * m_per, _silu(y))
            if d <= NEAR:
                y = jnp.dot(cwB_buf[d, :, :], w_ref[:, :],
                            preferred_element_type=jnp.float32)
                store_half(cw_origin * m_per + half, _silu(y))
            ccw_origin = lax.rem(my + d, N_DEV)
            y = jnp.dot(ccwB_buf[d, :, :], w_ref[:, :],
                        preferred_element_type=jnp.float32)
            store_half(ccw_origin * m_per + half, _silu(y))
            if d <= NEAR:
                y = jnp.dot(ccwA_buf[d, :, :], w_ref[:, :],
                            preferred_element_type=jnp.float32)
                store_half(ccw_origin * m_per, _silu(y))

        for h in range(FAR):
            for s in range(NSEG):
                cwA[h][s].wait_send()
                ccwB[h][s].wait_send()
                if h < NEAR:
                    cwB[h][s].wait_send()
                    ccwA[h][s].wait_send()
        for cp in pending:
            if cp is not None:
                cp.wait()

    return pl.pallas_call(
        body,
        out_shape=jax.ShapeDtypeStruct((N_DEV * m_per, n_per), jnp.float32),
        in_specs=[
            pl.BlockSpec(memory_space=pltpu.VMEM),
            pl.BlockSpec(memory_space=pltpu.VMEM),
        ],
        out_specs=pl.BlockSpec(memory_space=pl.ANY),
        scratch_shapes=[
            pltpu.VMEM((m_per, k), jnp.bfloat16),
            pltpu.VMEM((k, n_per), jnp.bfloat16),
            pltpu.VMEM((2, half, n_per), jnp.float32),
            pltpu.SemaphoreType.DMA((2,)),
            pltpu.VMEM((FAR + 1, half, k), jnp.bfloat16),
            pltpu.VMEM((NEAR + 1, half, k), jnp.bfloat16),
            pltpu.VMEM((FAR + 1, half, k), jnp.bfloat16),
            pltpu.VMEM((NEAR + 1, half, k), jnp.bfloat16),
            pltpu.SemaphoreType.DMA((FAR, NSEG)),
            pltpu.SemaphoreType.DMA((FAR, NSEG)),
            pltpu.SemaphoreType.DMA((NEAR, NSEG)),
            pltpu.SemaphoreType.DMA((NEAR, NSEG)),
            pltpu.SemaphoreType.DMA((FAR, NSEG)),
            pltpu.SemaphoreType.DMA((FAR, NSEG)),
            pltpu.SemaphoreType.DMA((NEAR, NSEG)),
            pltpu.SemaphoreType.DMA((NEAR, NSEG)),
        ],
        compiler_params=pltpu.CompilerParams(
            collective_id=0,
            vmem_limit_bytes=63 * 1024 * 1024,
        ),
    )(x, w_mat)
